# initial kernel scaffold (unmeasured)
import jax
import jax.numpy as jnp
from jax import lax
from jax.experimental import pallas as pl
from jax.experimental.pallas import tpu as pltpu

N_DEV = 4
MC = 1024
W = 2048
N_HOPS = 2 * (N_DEV - 1)


def kernel(x, w_mat, scale_x, scale_w):
    M, _ = x.shape
    _, N = w_mat.shape
    n_panels = N // W

    def body(x_ref, w_ref, sx_ref, sw_ref, out_ref,
             a_buf, b_bufs, send_sems, recv_sems, copy_sem):
        d = lax.axis_index("i")
        left = lax.rem(d + N_DEV - 1, N_DEV)
        right = lax.rem(d + 1, N_DEV)

        barrier_sem = pltpu.get_barrier_semaphore()
        for nbr in (left, right):
            pl.semaphore_signal(barrier_sem, inc=1, device_id=(nbr,),
                                device_id_type=pl.DeviceIdType.MESH)
        pl.semaphore_wait(barrier_sem, 2)

        scale = sx_ref[0] * sw_ref[0]

        def pchunk(c, cols):
            xa = x_ref[pl.ds(c * MC, MC), :]
            wb = w_ref[:, cols]
            return jnp.dot(xa, wb, preferred_element_type=jnp.float32)

        for p in range(n_panels):
            cols = pl.ds(p * W, W)

            a_buf[...] = pchunk(d, cols)
            for s in range(N_DEV - 1):
                rdma = pltpu.make_async_remote_copy(
                    src_ref=a_buf,
                    dst_ref=b_bufs.at[s],
                    send_sem=send_sems.at[s],
                    recv_sem=recv_sems.at[s],
                    device_id=(right,),
                    device_id_type=pl.DeviceIdType.MESH,
                )
                rdma.start()
                rdma.wait()
                c = lax.rem(d + N_DEV - s - 1, N_DEV)
                a_buf[...] = b_bufs[s] + pchunk(c, cols)

            y = a_buf[...] * scale
            a_buf[...] = y / (1.0 + jnp.exp(-jnp.clip(y, -60.0, 60.0)))
            r = lax.rem(d + 1, N_DEV)

            st = pltpu.make_async_copy(
                a_buf, out_ref.at[pl.ds(r * MC, MC), cols], copy_sem)
            st.start()
            st.wait()
            for t in range(N_DEV - 1):
                h = N_DEV - 1 + t
                src = a_buf if t == 0 else b_bufs.at[h - 1]
                rdma = pltpu.make_async_remote_copy(
                    src_ref=src,
                    dst_ref=b_bufs.at[h],
                    send_sem=send_sems.at[h],
                    recv_sem=recv_sems.at[h],
                    device_id=(right,),
                    device_id_type=pl.DeviceIdType.MESH,
                )
                rdma.start()
                rdma.wait()
                c = lax.rem(d + N_DEV - t, N_DEV)
                st = pltpu.make_async_copy(
                    b_bufs.at[h], out_ref.at[pl.ds(c * MC, MC), cols],
                    copy_sem)
                st.start()
                st.wait()

    return pl.pallas_call(
        body,
        out_shape=jax.ShapeDtypeStruct((M, N), jnp.float32),
        in_specs=[
            pl.BlockSpec(memory_space=pltpu.VMEM),
            pl.BlockSpec(memory_space=pltpu.VMEM),
            pl.BlockSpec(memory_space=pltpu.SMEM),
            pl.BlockSpec(memory_space=pltpu.SMEM),
        ],
        out_specs=pl.BlockSpec(memory_space=pltpu.ANY),
        scratch_shapes=[
            pltpu.VMEM((MC, W), jnp.float32),
            pltpu.VMEM((N_HOPS, MC, W), jnp.float32),
            pltpu.SemaphoreType.DMA((N_HOPS,)),
            pltpu.SemaphoreType.DMA((N_HOPS,)),
            pltpu.SemaphoreType.DMA,
        ],
        compiler_params=pltpu.CompilerParams(collective_id=0),
    )(x, w_mat, scale_x, scale_w)


# baseline (device time: 2455645 ns/iter reference)
import jax
import jax.numpy as jnp
from jax import lax
from jax.experimental import pallas as pl
from jax.experimental.pallas import tpu as pltpu

N_DEV = 4
MC = 1024
W = 1024
N_HOPS = 2 * (N_DEV - 1)


def kernel(x, w_mat, scale_x, scale_w):
    M, _ = x.shape
    _, N = w_mat.shape
    n_panels = N // W
    x = x.astype(jnp.float8_e5m2)
    w_mat = w_mat.astype(jnp.float8_e5m2)

    def body(x_ref, w_ref, sx_ref, sw_ref, out_ref,
             a_buf, b_bufs, send_sems, recv_sems, copy_sem):
        d = lax.axis_index("i")
        left = lax.rem(d + N_DEV - 1, N_DEV)
        right = lax.rem(d + 1, N_DEV)

        barrier_sem = pltpu.get_barrier_semaphore()
        for nbr in (left, right):
            pl.semaphore_signal(barrier_sem, inc=1, device_id=(nbr,),
                                device_id_type=pl.DeviceIdType.MESH)
        pl.semaphore_wait(barrier_sem, 2)

        scale = sx_ref[0] * sw_ref[0]

        def pchunk(c, cols):
            xa = x_ref[pl.ds(c * MC, MC), :]
            wb = w_ref[:, cols]
            return jnp.dot(xa, wb, preferred_element_type=jnp.float32)

        for p in range(n_panels):
            cols = pl.ds(p * W, W)

            a_buf[...] = pchunk(d, cols)
            for s in range(N_DEV - 1):
                rdma = pltpu.make_async_remote_copy(
                    src_ref=a_buf,
                    dst_ref=b_bufs.at[s],
                    send_sem=send_sems.at[s],
                    recv_sem=recv_sems.at[s],
                    device_id=(right,),
                    device_id_type=pl.DeviceIdType.MESH,
                )
                rdma.start()
                rdma.wait()
                c = lax.rem(d + N_DEV - s - 1, N_DEV)
                a_buf[...] = b_bufs[s] + pchunk(c, cols)

            y = a_buf[...] * scale
            a_buf[...] = y / (1.0 + jnp.exp(-jnp.clip(y, -60.0, 60.0)))
            r = lax.rem(d + 1, N_DEV)

            st = pltpu.make_async_copy(
                a_buf, out_ref.at[pl.ds(r * MC, MC), cols], copy_sem)
            st.start()
            st.wait()
            for t in range(N_DEV - 1):
                h = N_DEV - 1 + t
                src = a_buf if t == 0 else b_bufs.at[h - 1]
                rdma = pltpu.make_async_remote_copy(
                    src_ref=src,
                    dst_ref=b_bufs.at[h],
                    send_sem=send_sems.at[h],
                    recv_sem=recv_sems.at[h],
                    device_id=(right,),
                    device_id_type=pl.DeviceIdType.MESH,
                )
                rdma.start()
                rdma.wait()
                c = lax.rem(d + N_DEV - t, N_DEV)
                st = pltpu.make_async_copy(
                    b_bufs.at[h], out_ref.at[pl.ds(c * MC, MC), cols],
                    copy_sem)
                st.start()
                st.wait()

    return pl.pallas_call(
        body,
        out_shape=jax.ShapeDtypeStruct((M, N), jnp.float32),
        in_specs=[
            pl.BlockSpec(memory_space=pltpu.VMEM),
            pl.BlockSpec(memory_space=pltpu.VMEM),
            pl.BlockSpec(memory_space=pltpu.SMEM),
            pl.BlockSpec(memory_space=pltpu.SMEM),
        ],
        out_specs=pl.BlockSpec(memory_space=pl.ANY),
        scratch_shapes=[
            pltpu.VMEM((MC, W), jnp.float32),
            pltpu.VMEM((N_HOPS, MC, W), jnp.float32),
            pltpu.SemaphoreType.DMA((N_HOPS,)),
            pltpu.SemaphoreType.DMA((N_HOPS,)),
            pltpu.SemaphoreType.DMA,
        ],
        compiler_params=pltpu.CompilerParams(collective_id=0),
    )(x, w_mat, scale_x, scale_w)


# device time: 799570 ns/iter; 3.0712x vs baseline; 3.0712x over previous
import jax
import jax.numpy as jnp
from jax import lax
from jax.experimental import pallas as pl
from jax.experimental.pallas import tpu as pltpu

N_DEV = 4
MC = 1024
W = 1024
N_SLOTS = 4


def kernel(x, w_mat, scale_x, scale_w):
    M, _ = x.shape
    _, N = w_mat.shape
    n_pairs = N // (2 * W)
    x = x.astype(jnp.float8_e5m2)
    w_mat = w_mat.astype(jnp.float8_e5m2)

    def body(x_ref, w_ref, sx_ref, sw_ref, out_ref,
             a_bufs, b_bufs, stage, send_sems, recv_sems, store_sem):
        d = lax.axis_index("i")
        left = lax.rem(d + N_DEV - 1, N_DEV)
        right = lax.rem(d + 1, N_DEV)

        barrier_sem = pltpu.get_barrier_semaphore()
        for nbr in (left, right):
            pl.semaphore_signal(barrier_sem, inc=1, device_id=(nbr,),
                                device_id_type=pl.DeviceIdType.MESH)
        pl.semaphore_wait(barrier_sem, 2)

        scale = sx_ref[0] * sw_ref[0]

        def pchunk(c, p):
            xa = x_ref[pl.ds(c * MC, MC), :]
            wb = w_ref[:, pl.ds(p * W, W)]
            return jnp.dot(xa, wb, preferred_element_type=jnp.float32)

        def hop(di, src, slot, dst_dev):
            return pltpu.make_async_remote_copy(
                src_ref=src,
                dst_ref=b_bufs.at[di, slot],
                send_sem=send_sems.at[di, slot],
                recv_sem=recv_sems.at[di, slot],
                device_id=(dst_dev,),
                device_id_type=pl.DeviceIdType.MESH,
            )

        def store(src_bf16_or_f32, c, p):
            stage[...] = src_bf16_or_f32.astype(jnp.float32)
            st = pltpu.make_async_copy(
                stage, out_ref.at[pl.ds(c * MC, MC), pl.ds(p * W, W)],
                store_sem)
            st.start()
            st.wait()

        for pp in range(n_pairs):
            pR = 2 * pp
            pL = 2 * pp + 1

            a_bufs[0] = pchunk(d, pR).astype(jnp.bfloat16)
            a_bufs[1] = pchunk(d, pL).astype(jnp.bfloat16)
            for s in range(N_DEV - 1):
                slot = (6 * pp + s) % N_SLOTS
                rdR = hop(0, a_bufs.at[0], slot, right)
                rdL = hop(1, a_bufs.at[1], slot, left)
                rdR.start()
                rdL.start()
                cR = lax.rem(d + N_DEV - s - 1, N_DEV)
                cL = lax.rem(d + s + 1, N_DEV)
                nxtR = pchunk(cR, pR)
                nxtL = pchunk(cL, pL)
                rdR.wait()
                rdL.wait()
                a_bufs[0] = (b_bufs[0, slot] + nxtR).astype(jnp.bfloat16)
                a_bufs[1] = (b_bufs[1, slot] + nxtL).astype(jnp.bfloat16)

            rown = (lax.rem(d + 1, N_DEV), lax.rem(d + N_DEV - 1, N_DEV))
            for di, p in ((0, pR), (1, pL)):
                y = a_bufs[di].astype(jnp.float32) * scale
                act = y / (1.0 + jnp.exp(-jnp.clip(y, -60.0, 60.0)))
                a_bufs[di] = act.astype(jnp.bfloat16)
                store(act, rown[di], p)

            for t in range(N_DEV - 1):
                g = 6 * pp + N_DEV - 1 + t
                slot = g % N_SLOTS
                prev = (g - 1) % N_SLOTS
                srcR = a_bufs.at[0] if t == 0 else b_bufs.at[0, prev]
                srcL = a_bufs.at[1] if t == 0 else b_bufs.at[1, prev]
                rdR = hop(0, srcR, slot, right)
                rdL = hop(1, srcL, slot, left)
                rdR.start()
                rdL.start()
                rdR.wait()
                rdL.wait()
                store(b_bufs[0, slot], lax.rem(d + N_DEV - t, N_DEV), pR)
                store(b_bufs[1, slot], lax.rem(d + t, N_DEV), pL)

    return pl.pallas_call(
        body,
        out_shape=jax.ShapeDtypeStruct((M, N), jnp.float32),
        in_specs=[
            pl.BlockSpec(memory_space=pltpu.VMEM),
            pl.BlockSpec(memory_space=pltpu.VMEM),
            pl.BlockSpec(memory_space=pltpu.SMEM),
            pl.BlockSpec(memory_space=pltpu.SMEM),
        ],
        out_specs=pl.BlockSpec(memory_space=pl.ANY),
        scratch_shapes=[
            pltpu.VMEM((2, MC, W), jnp.bfloat16),
            pltpu.VMEM((2, N_SLOTS, MC, W), jnp.bfloat16),
            pltpu.VMEM((MC, W), jnp.float32),
            pltpu.SemaphoreType.DMA((2, N_SLOTS)),
            pltpu.SemaphoreType.DMA((2, N_SLOTS)),
            pltpu.SemaphoreType.DMA,
        ],
        compiler_params=pltpu.CompilerParams(
            collective_id=0,
            vmem_limit_bytes=64 * 1024 * 1024,
        ),
    )(x, w_mat, scale_x, scale_w)


# device time: 672625 ns/iter; 3.6508x vs baseline; 1.1887x over previous
import jax
import jax.numpy as jnp
from jax import lax
from jax.experimental import pallas as pl
from jax.experimental.pallas import tpu as pltpu

N_DEV = 4
MC = 1024
W = 1024
N_CH = 2
N_SLOT = 2
N_GRP = 2


def kernel(x, w_mat, scale_x, scale_w):
    M, _ = x.shape
    _, N = w_mat.shape
    x = x.astype(jnp.float8_e5m2)
    w_mat = w_mat.astype(jnp.float8_e5m2)

    def body(x_ref, w_ref, sx_ref, sw_ref, out_ref,
             a_bufs, b_bufs, stage, send_sems, recv_sems, credit_sems,
             store_sem):
        d = lax.axis_index("i")
        left = lax.rem(d + N_DEV - 1, N_DEV)
        right = lax.rem(d + 1, N_DEV)

        barrier_sem = pltpu.get_barrier_semaphore()
        for nbr in (left, right):
            pl.semaphore_signal(barrier_sem, inc=1, device_id=(nbr,),
                                device_id_type=pl.DeviceIdType.MESH)
        pl.semaphore_wait(barrier_sem, 2)

        scale = sx_ref[0] * sw_ref[0]
        dst = (right, left)
        ups = (left, right)

        def pnl(g, ch, di):
            return 4 * g + 2 * ch + di

        def pchunk(c, p):
            xa = x_ref[pl.ds(c * MC, MC), :]
            wb = w_ref[:, pl.ds(p * W, W)]
            return jnp.dot(xa, wb, preferred_element_type=jnp.float32)

        def start_hop(ch, h, srcs, need_credit):
            descs = []
            for di in range(2):
                if need_credit:
                    pl.semaphore_wait(credit_sems.at[di, ch], 1)
                rd = pltpu.make_async_remote_copy(
                    src_ref=srcs[di],
                    dst_ref=b_bufs.at[di, ch, h % N_SLOT],
                    send_sem=send_sems.at[di, ch, h % N_SLOT],
                    recv_sem=recv_sems.at[di, ch, h % N_SLOT],
                    device_id=(dst[di],),
                    device_id_type=pl.DeviceIdType.MESH,
                )
                rd.start()
                descs.append(rd)
            return descs

        def send_credits(ch):
            for di in range(2):
                pl.semaphore_signal(credit_sems.at[di, ch], inc=1,
                                    device_id=(ups[di],),
                                    device_id_type=pl.DeviceIdType.MESH)

        def store(val_f32, c, p):
            stage[...] = val_f32
            st = pltpu.make_async_copy(
                stage, out_ref.at[pl.ds(c * MC, MC), pl.ds(p * W, W)],
                store_sem)
            st.start()
            st.wait()

        own = (lax.rem(d + 1, N_DEV), lax.rem(d + N_DEV - 1, N_DEV))

        infl = {}
        for g in range(N_GRP):
            for ch in range(N_CH):
                a_bufs[0, ch] = pchunk(d, pnl(g, ch, 0)).astype(jnp.bfloat16)
                a_bufs[1, ch] = pchunk(d, pnl(g, ch, 1)).astype(jnp.bfloat16)
                infl[ch] = start_hop(
                    ch, 0, (a_bufs.at[0, ch], a_bufs.at[1, ch]),
                    need_credit=(g > 0))

            for h in range(6):
                for ch in range(N_CH):
                    if h < 3:
                        nxt = (
                            pchunk(lax.rem(d + N_DEV - h - 1, N_DEV),
                                   pnl(g, ch, 0)),
                            pchunk(lax.rem(d + h + 1, N_DEV),
                                   pnl(g, ch, 1)),
                        )
                    for rd in infl[ch]:
                        rd.wait()

                    slot = h % N_SLOT
                    if h < 2:
                        for di in range(2):
                            a_bufs[di, ch] = (
                                b_bufs[di, ch, slot] + nxt[di]
                            ).astype(jnp.bfloat16)
                        send_credits(ch)
                        infl[ch] = start_hop(
                            ch, h + 1,
                            (a_bufs.at[0, ch], a_bufs.at[1, ch]),
                            need_credit=(6 * g + h + 1 >= 2))
                    elif h == 2:
                        acts = []
                        for di in range(2):
                            y = (b_bufs[di, ch, slot] + nxt[di]) * scale
                            act = y / (1.0 + jnp.exp(-jnp.clip(y, -60.0,
                                                               60.0)))
                            a_bufs[di, ch] = act.astype(jnp.bfloat16)
                            acts.append(act)
                        send_credits(ch)
                        infl[ch] = start_hop(
                            ch, 3, (a_bufs.at[0, ch], a_bufs.at[1, ch]),
                            need_credit=True)
                        for di in range(2):
                            store(acts[di], own[di], pnl(g, ch, di))
                    else:
                        t = h - 3
                        if h == 4:
                            send_credits(ch)
                        if h == 5 and g < N_GRP - 1:
                            send_credits(ch)
                        if h < 5:
                            infl[ch] = start_hop(
                                ch, h + 1,
                                (b_bufs.at[0, ch, slot],
                                 b_bufs.at[1, ch, slot]),
                                need_credit=True)
                        rows = (lax.rem(d + N_DEV - t, N_DEV),
                                lax.rem(d + t, N_DEV))
                        for di in range(2):
                            store(b_bufs[di, ch, slot].astype(jnp.float32),
                                  rows[di], pnl(g, ch, di))
                        if h == 5 and g < N_GRP - 1:
                            send_credits(ch)

    return pl.pallas_call(
        body,
        out_shape=jax.ShapeDtypeStruct((M, N), jnp.float32),
        in_specs=[
            pl.BlockSpec(memory_space=pltpu.VMEM),
            pl.BlockSpec(memory_space=pltpu.VMEM),
            pl.BlockSpec(memory_space=pltpu.SMEM),
            pl.BlockSpec(memory_space=pltpu.SMEM),
        ],
        out_specs=pl.BlockSpec(memory_space=pl.ANY),
        scratch_shapes=[
            pltpu.VMEM((2, N_CH, MC, W), jnp.bfloat16),
            pltpu.VMEM((2, N_CH, N_SLOT, MC, W), jnp.bfloat16),
            pltpu.VMEM((MC, W), jnp.float32),
            pltpu.SemaphoreType.DMA((2, N_CH, N_SLOT)),
            pltpu.SemaphoreType.DMA((2, N_CH, N_SLOT)),
            pltpu.SemaphoreType.REGULAR((2, N_CH)),
            pltpu.SemaphoreType.DMA,
        ],
        compiler_params=pltpu.CompilerParams(
            collective_id=0,
            vmem_limit_bytes=64 * 1024 * 1024,
        ),
    )(x, w_mat, scale_x, scale_w)
